# baseline (speedup 1.0000x reference)
import jax
import jax.numpy as jnp
from jax import lax
from jax.experimental import pallas as pl
from jax.experimental.pallas import tpu as pltpu
from jax.experimental.pallas import tpu_sc as plsc

_MAXN = 5
_K = 8
_B = 64
_L = 8192
_PAD = 16
_BUF = _PAD + _L + 16
_INF = 1 << 30
_GROUP = 16
_LOW = 0xFFFF


def _row_scan(buf, s, idx16):
    end = s - _K
    tails = plsc.load_gather(buf, [jnp.maximum(_PAD + s - 1 - idx16, 0)])
    tails = tails & _LOW
    t = [jnp.max(jnp.where(idx16 == i, tails, 0)) for i in range(_MAXN)]
    t01 = t[0] | (t[1] << 16)
    t23 = t[2] | (t[3] << 16)

    def fast_group(base):
        acc = None
        for g in range(_GROUP):
            m = buf[pl.ds(_PAD + base + g * 16, 16)] == t01
            acc = m if acc is None else (acc | m)
        return plsc.all_reduce_population_count(acc)[0] > 0

    def slow_group(base, es):
        def one_chunk(g, es):
            off = _PAD + base + g * 16
            pos = base + g * 16 + idx16
            p0 = buf[pl.ds(off, 16)]
            p2 = buf[pl.ds(off - 2, 16)]
            p4 = buf[pl.ds(off - 4, 16)]
            m2 = (p0 == t01) & (pos < end)
            m3 = m2 & ((p2 & _LOW) == t[2])
            m4 = m2 & (p2 == t23)
            m5 = m4 & ((p4 & _LOW) == t[4])
            e2, e3, e4, e5 = es
            e2 = jnp.minimum(e2, jnp.min(jnp.where(m2, pos, _INF)))
            e3 = jnp.minimum(e3, jnp.min(jnp.where(m3, pos, _INF)))
            e4 = jnp.minimum(e4, jnp.min(jnp.where(m4, pos, _INF)))
            e5 = jnp.minimum(e5, jnp.min(jnp.where(m5, pos, _INF)))
            return e2, e3, e4, e5
        return lax.fori_loop(0, _GROUP, one_chunk, es)

    def cond(c):
        return (c[0] < end) & (c[4] >= _INF)

    def body(c):
        base = c[0]
        es = c[1:]
        hit = fast_group(base)
        es = lax.cond(hit, lambda: slow_group(base, es), lambda: es)
        return (base + _GROUP * 16,) + es

    inf = jnp.int32(_INF)
    out = lax.while_loop(cond, body, (jnp.int32(0), inf, inf, inf, inf))
    return out[1:]


def _make_body(num_cores, n_workers):
    rows_per_worker = _B // n_workers

    def body(nums_hbm, toks_hbm, out_hbm,
             nums_v, buf0, buf1, stage, sem0, sem1):
        wid = lax.axis_index("s") * num_cores + lax.axis_index("c")
        idx16 = lax.iota(jnp.int32, 16)
        pltpu.sync_copy(nums_hbm, nums_v)

        bufs = (buf0, buf1)
        sems = (sem0, sem1)
        copies = []
        for r in range(rows_per_worker):
            row = wid * rows_per_worker + r
            bufs[r][pl.ds(0, 16)] = jnp.full((16,), -1, jnp.int32)
            copies.append(pltpu.async_copy(
                toks_hbm.at[row], bufs[r].at[pl.ds(_PAD, _L)], sems[r]))

        for r in range(rows_per_worker):
            row = wid * rows_per_worker + r
            copies[r].wait()
            buf = bufs[r]
            base16 = (row // 16) * 16
            lane = row - base16
            s = jnp.max(jnp.where(idx16 == lane, nums_v[pl.ds(base16, 16)], 0))
            e2, e3, e4, e5 = _row_scan(buf, s, idx16)
            best = jnp.where(e5 < _INF, e5,
                             jnp.where(e4 < _INF, e4,
                                       jnp.where(e3 < _INF, e3, e2)))
            has = best < _INF
            start = jnp.where(has, best + 1, 0)
            ext = buf[pl.ds(_PAD + start, 16)] & _LOW
            stage[...] = jnp.where(has & (idx16 < _K), ext, 0)
            pltpu.sync_copy(stage.at[pl.ds(0, _K)],
                            out_hbm.at[pl.ds(row * _K, _K)])

    return body


def kernel(num_tokens_no_spec, token_ids_gpu, combined_mask):
    s_eff = jnp.where(combined_mask, num_tokens_no_spec, 0).astype(jnp.int32)
    prev = jnp.pad(token_ids_gpu, ((0, 0), (1, 0)),
                   constant_values=-1)[:, :-1]
    pair = token_ids_gpu | (prev << 16)
    mesh = plsc.VectorSubcoreMesh(core_axis_name="c", subcore_axis_name="s")
    n_workers = mesh.num_cores * mesh.num_subcores
    out = pl.kernel(
        _make_body(mesh.num_cores, n_workers),
        out_type=jax.ShapeDtypeStruct((_B * _K,), jnp.int32),
        mesh=mesh,
        compiler_params=pltpu.CompilerParams(
            needs_layout_passes=False, use_tc_tiling_on_sc=False),
        scratch_types=[
            pltpu.VMEM((_B,), jnp.int32),
            pltpu.VMEM((_BUF,), jnp.int32),
            pltpu.VMEM((_BUF,), jnp.int32),
            pltpu.VMEM((16,), jnp.int32),
            pltpu.SemaphoreType.DMA,
            pltpu.SemaphoreType.DMA,
        ],
    )(s_eff, pair)
    return out.reshape(_B, _K)


# speedup vs baseline: 1.0245x; 1.0245x over previous
import jax
import jax.numpy as jnp
from jax import lax
from jax.experimental import pallas as pl
from jax.experimental.pallas import tpu as pltpu
from jax.experimental.pallas import tpu_sc as plsc

_MAXN = 5
_K = 8
_B = 64
_L = 8192
_PAD = 16
_BUF = _PAD + _L + 16
_INF = 1 << 30
_GROUP = 16
_LOW = 0xFFFF


def _row_scan(buf, s, idx16):
    end = s - _K
    tails = plsc.load_gather(buf, [jnp.maximum(_PAD + s - 1 - idx16, 0)])
    tails = tails & _LOW
    t = [jnp.max(jnp.where(idx16 == i, tails, 0)) for i in range(_MAXN)]
    t01 = t[0] | (t[1] << 16)
    t23 = t[2] | (t[3] << 16)

    def fast_group(base):
        acc = None
        for g in range(_GROUP):
            m = buf[pl.ds(_PAD + base + g * 16, 16)] == t01
            acc = m if acc is None else (acc | m)
        return plsc.all_reduce_population_count(acc)[0] > 0

    def slow_group(base, es):
        def one_chunk(g, es):
            off = _PAD + base + g * 16
            pos = base + g * 16 + idx16
            p0 = buf[pl.ds(off, 16)]
            p2 = buf[pl.ds(off - 2, 16)]
            p4 = buf[pl.ds(off - 4, 16)]
            m2 = (p0 == t01) & (pos < end)
            m3 = m2 & ((p2 & _LOW) == t[2])
            m4 = m2 & (p2 == t23)
            m5 = m4 & ((p4 & _LOW) == t[4])
            e2, e3, e4, e5 = es
            e2 = jnp.minimum(e2, jnp.min(jnp.where(m2, pos, _INF)))
            e3 = jnp.minimum(e3, jnp.min(jnp.where(m3, pos, _INF)))
            e4 = jnp.minimum(e4, jnp.min(jnp.where(m4, pos, _INF)))
            e5 = jnp.minimum(e5, jnp.min(jnp.where(m5, pos, _INF)))
            return e2, e3, e4, e5
        return lax.fori_loop(0, _GROUP, one_chunk, es)

    def cond(c):
        return (c[0] < end) & (c[4] >= _INF)

    def body(c):
        base = c[0]
        es = c[1:]
        hit = fast_group(base)
        es = lax.cond(hit, lambda: slow_group(base, es), lambda: es)
        return (base + _GROUP * 16,) + es

    inf = jnp.int32(_INF)
    out = lax.while_loop(cond, body, (jnp.int32(0), inf, inf, inf, inf))
    return out[1:]


def _make_body(num_cores, n_workers):
    rows_per_worker = _B // n_workers

    def body(nums_hbm, toks_hbm, out_hbm,
             nums_v, buf0, buf1, stage, sem0, sem1):
        wid = lax.axis_index("s") * num_cores + lax.axis_index("c")
        idx16 = lax.iota(jnp.int32, 16)
        pltpu.sync_copy(nums_hbm, nums_v)

        bufs = (buf0, buf1)
        sems = (sem0, sem1)
        copies = []
        for r in range(rows_per_worker):
            row = wid * rows_per_worker + r
            bufs[r][pl.ds(0, 16)] = jnp.full((16,), -1, jnp.int32)
            copies.append(pltpu.async_copy(
                toks_hbm.at[pl.ds(row * _L, _L)],
                bufs[r].at[pl.ds(_PAD, _L)], sems[r]))

        for r in range(rows_per_worker):
            row = wid * rows_per_worker + r
            copies[r].wait()
            buf = bufs[r]
            base16 = (row // 16) * 16
            lane = row - base16
            s = jnp.max(jnp.where(idx16 == lane, nums_v[pl.ds(base16, 16)], 0))
            e2, e3, e4, e5 = _row_scan(buf, s, idx16)
            best = jnp.where(e5 < _INF, e5,
                             jnp.where(e4 < _INF, e4,
                                       jnp.where(e3 < _INF, e3, e2)))
            has = best < _INF
            start = jnp.where(has, best + 1, 0)
            ext = buf[pl.ds(_PAD + start, 16)] & _LOW
            stage[...] = jnp.where(has & (idx16 < _K), ext, 0)
            pltpu.sync_copy(stage.at[pl.ds(0, _K)],
                            out_hbm.at[pl.ds(row * _K, _K)])

    return body


def kernel(num_tokens_no_spec, token_ids_gpu, combined_mask):
    s_eff = jnp.where(combined_mask, num_tokens_no_spec, 0).astype(jnp.int32)
    flat = token_ids_gpu.reshape(_B * _L)
    prev = jnp.concatenate([jnp.full((1,), -1, jnp.int32), flat[:-1]])
    row_start = (jax.lax.iota(jnp.int32, _B * _L) % _L) == 0
    pair = flat | (jnp.where(row_start, -1, prev) << 16)
    mesh = plsc.VectorSubcoreMesh(core_axis_name="c", subcore_axis_name="s")
    n_workers = mesh.num_cores * mesh.num_subcores
    out = pl.kernel(
        _make_body(mesh.num_cores, n_workers),
        out_type=jax.ShapeDtypeStruct((_B * _K,), jnp.int32),
        mesh=mesh,
        compiler_params=pltpu.CompilerParams(
            needs_layout_passes=False, use_tc_tiling_on_sc=False),
        scratch_types=[
            pltpu.VMEM((_B,), jnp.int32),
            pltpu.VMEM((_BUF,), jnp.int32),
            pltpu.VMEM((_BUF,), jnp.int32),
            pltpu.VMEM((16,), jnp.int32),
            pltpu.SemaphoreType.DMA,
            pltpu.SemaphoreType.DMA,
        ],
    )(s_eff, pair)
    return out.reshape(_B, _K)


# speedup vs baseline: 1.0744x; 1.0487x over previous
import jax
import jax.numpy as jnp
from jax import lax
from jax.experimental import pallas as pl
from jax.experimental.pallas import tpu as pltpu
from jax.experimental.pallas import tpu_sc as plsc

_MAXN = 5
_K = 8
_B = 64
_L = 8192
_PAD = 16
_BUF = _PAD + _L + 16
_INF = 1 << 30
_GROUP = 32
_LOW = 0xFFFF


def _row_scan(buf, s, idx16):
    end = s - _K
    tails = plsc.load_gather(buf, [jnp.maximum(_PAD + s - 1 - idx16, 0)])
    tails = tails & _LOW
    t = [jnp.max(jnp.where(idx16 == i, tails, 0)) for i in range(_MAXN)]
    t01 = t[0] | (t[1] << 16)
    t23 = t[2] | (t[3] << 16)

    def fast_group(base):
        acc = None
        for g in range(_GROUP):
            m = buf[pl.ds(_PAD + base + g * 16, 16)] == t01
            acc = m if acc is None else (acc | m)
        return plsc.all_reduce_population_count(acc)[0] > 0

    def slow_group(base, es):
        def one_chunk(g, es):
            off = _PAD + base + g * 16
            pos = base + g * 16 + idx16
            p0 = buf[pl.ds(off, 16)]
            p2 = buf[pl.ds(off - 2, 16)]
            p4 = buf[pl.ds(off - 4, 16)]
            m2 = (p0 == t01) & (pos < end)
            m3 = m2 & ((p2 & _LOW) == t[2])
            m4 = m2 & (p2 == t23)
            m5 = m4 & ((p4 & _LOW) == t[4])
            e2, e3, e4, e5 = es
            e2 = jnp.minimum(e2, jnp.min(jnp.where(m2, pos, _INF)))
            e3 = jnp.minimum(e3, jnp.min(jnp.where(m3, pos, _INF)))
            e4 = jnp.minimum(e4, jnp.min(jnp.where(m4, pos, _INF)))
            e5 = jnp.minimum(e5, jnp.min(jnp.where(m5, pos, _INF)))
            return e2, e3, e4, e5
        return lax.fori_loop(0, _GROUP, one_chunk, es)

    def cond(c):
        return (c[0] < end) & (c[4] >= _INF)

    def body(c):
        base = c[0]
        es = c[1:]
        hit = fast_group(base)
        es = lax.cond(hit, lambda: slow_group(base, es), lambda: es)
        return (base + _GROUP * 16,) + es

    inf = jnp.int32(_INF)
    out = lax.while_loop(cond, body, (jnp.int32(0), inf, inf, inf, inf))
    return out[1:]


def _make_body(num_cores, n_workers):
    rows_per_worker = _B // n_workers

    def body(nums_hbm, toks_hbm, out_hbm,
             nums_v, buf0, buf1, stage, sem0, sem1, semn):
        wid = lax.axis_index("s") * num_cores + lax.axis_index("c")
        idx16 = lax.iota(jnp.int32, 16)
        nums_cp = pltpu.async_copy(nums_hbm, nums_v, semn)

        bufs = (buf0, buf1)
        sems = (sem0, sem1)
        copies = []
        for r in range(rows_per_worker):
            row = wid * rows_per_worker + r
            bufs[r][pl.ds(0, 16)] = jnp.full((16,), -1, jnp.int32)
            copies.append(pltpu.async_copy(
                toks_hbm.at[pl.ds(row * _L, _L)],
                bufs[r].at[pl.ds(_PAD, _L)], sems[r]))
        nums_cp.wait()

        for r in range(rows_per_worker):
            row = wid * rows_per_worker + r
            copies[r].wait()
            buf = bufs[r]
            base16 = (row // 16) * 16
            lane = row - base16
            s = jnp.max(jnp.where(idx16 == lane, nums_v[pl.ds(base16, 16)], 0))
            e2, e3, e4, e5 = _row_scan(buf, s, idx16)
            best = jnp.where(e5 < _INF, e5,
                             jnp.where(e4 < _INF, e4,
                                       jnp.where(e3 < _INF, e3, e2)))
            has = best < _INF
            start = jnp.where(has, best + 1, 0)
            ext = buf[pl.ds(_PAD + start, 16)] & _LOW
            stage[...] = jnp.where(has & (idx16 < _K), ext, 0)
            pltpu.sync_copy(stage.at[pl.ds(0, _K)],
                            out_hbm.at[pl.ds(row * _K, _K)])

    return body


def kernel(num_tokens_no_spec, token_ids_gpu, combined_mask):
    s_eff = jnp.where(combined_mask, num_tokens_no_spec, 0).astype(jnp.int32)
    flat = token_ids_gpu.reshape(_B * _L)
    prev = jnp.concatenate([jnp.full((1,), -1, jnp.int32), flat[:-1]])
    row_start = (jax.lax.iota(jnp.int32, _B * _L) % _L) == 0
    pair = flat | (jnp.where(row_start, -1, prev) << 16)
    mesh = plsc.VectorSubcoreMesh(core_axis_name="c", subcore_axis_name="s")
    n_workers = mesh.num_cores * mesh.num_subcores
    out = pl.kernel(
        _make_body(mesh.num_cores, n_workers),
        out_type=jax.ShapeDtypeStruct((_B * _K,), jnp.int32),
        mesh=mesh,
        compiler_params=pltpu.CompilerParams(
            needs_layout_passes=False, use_tc_tiling_on_sc=False),
        scratch_types=[
            pltpu.VMEM((_B,), jnp.int32),
            pltpu.VMEM((_BUF,), jnp.int32),
            pltpu.VMEM((_BUF,), jnp.int32),
            pltpu.VMEM((16,), jnp.int32),
            pltpu.SemaphoreType.DMA,
            pltpu.SemaphoreType.DMA,
            pltpu.SemaphoreType.DMA,
        ],
    )(s_eff, pair)
    return out.reshape(_B, _K)
